# baseline (speedup 1.0000x reference)
import functools

import jax
import jax.numpy as jnp
from jax import lax
from jax.experimental import pallas as pl
from jax.experimental.pallas import tpu as pltpu
from jax.experimental.pallas import tpu_sc as plsc

B, N, NPOINT, SP_NUM = 2, 8192, 1024, 64
K1, K2, K3, K4 = 32, 16, 32, 16
C_IN, C1, D1, D2, CW2, CP, C_OUT = 32, 64, 16, 16, 96, 64, 128
CG1 = D1 + C1

NW = 32


def _fps_kernel_body(npoint, rows, cols):
    irows = max(npoint // 128, 1)

    def body(xyz_ref, out_ref):
        v = xyz_ref[...]
        flat = (
            lax.broadcasted_iota(jnp.int32, (rows, cols), 0) * cols
            + lax.broadcasted_iota(jnp.int32, (rows, cols), 1)
        )[None]
        pos = (
            lax.broadcasted_iota(jnp.int32, (irows, 128), 0) * 128
            + lax.broadcasted_iota(jnp.int32, (irows, 128), 1)
        )[None]

        def step(i, state):
            idxs, dists, far = state
            idxs = jnp.where(pos == i, far, idxs)
            sel = flat == far
            fxyz = jnp.sum(
                jnp.where(sel[:, None], v, 0.0), axis=(2, 3), keepdims=True
            )
            d = (
                (v[:, 0] - fxyz[:, 0]) ** 2
                + (v[:, 1] - fxyz[:, 1]) ** 2
                + (v[:, 2] - fxyz[:, 2]) ** 2
            )
            dists = jnp.minimum(dists, d)
            far2 = jnp.argmax(
                dists.reshape(B, rows * cols), axis=1
            ).astype(jnp.int32)[:, None, None]
            return (idxs, dists, far2)

        idxs0 = jnp.zeros((B, irows, 128), jnp.int32)
        d0 = jnp.full((B, rows, cols), 1e10, jnp.float32)
        far0 = jnp.zeros((B, 1, 1), jnp.int32)
        idxs, _, _ = lax.fori_loop(0, npoint, step, (idxs0, d0, far0))
        boff = lax.broadcasted_iota(jnp.int32, (B, irows, 128), 0) * (rows * cols)
        out_ref[...] = idxs + boff

    return body, irows


def _fps(xyz, npoint):
    n = xyz.shape[1]
    cols = 1024 if n >= 8192 else 128
    rows = n // cols
    xyz_t = jnp.transpose(xyz, (0, 2, 1)).reshape(B, 3, rows, cols)
    body, irows = _fps_kernel_body(npoint, rows, cols)
    out = pl.pallas_call(
        body,
        grid=(1,),
        in_specs=[pl.BlockSpec((B, 3, rows, cols), lambda i: (0, 0, 0, 0))],
        out_specs=pl.BlockSpec((B, irows, 128), lambda i: (0, 0, 0)),
        out_shape=jax.ShapeDtypeStruct((B, irows, 128), jnp.int32),
    )(xyz_t)
    return out.reshape(B, irows * 128)[:, :npoint]


def _knn_body(k, masked):
    def body(q_ref, r_ref, out_ref):
        q = q_ref[0]
        qb = q.shape[0]
        nr = r_ref.shape[2]
        rx = r_ref[0, 0:1, :]
        ry = r_ref[0, 1:2, :]
        rz = r_ref[0, 2:3, :]
        d = (q[:, 0:1] - rx) ** 2 + (q[:, 1:2] - ry) ** 2 + (q[:, 2:3] - rz) ** 2
        if masked:
            rc = r_ref[0, 3:4, :]
            d = d + 1e9 * (q[:, 3:4] != rc).astype(jnp.float32)
        col = lax.broadcasted_iota(jnp.int32, (qb, nr), 1)
        kcol = lax.broadcasted_iota(jnp.int32, (qb, k), 1)

        def ext(j, state):
            dd, idxs, prev = state
            dd = jnp.where(col == prev, jnp.inf, dd)
            idx = jnp.argmin(dd, axis=1).astype(jnp.int32)[:, None]
            idxs = jnp.where(kcol == j, idx, idxs)
            return dd, idxs, idx

        _, idxs, _ = lax.fori_loop(
            0, k, ext,
            (d, jnp.zeros((qb, k), jnp.int32), jnp.full((qb, 1), -1, jnp.int32)))
        out_ref[0] = idxs + pl.program_id(0) * nr

    return body


def _knn(query, ref, k, qcomp=None, rcomp=None, qb=128):
    b, nq, _ = query.shape
    nr = ref.shape[1]
    masked = qcomp is not None
    qc = qcomp.astype(jnp.float32) if masked else jnp.zeros((b, nq), jnp.float32)
    rc = rcomp.astype(jnp.float32) if masked else jnp.zeros((b, nr), jnp.float32)
    q4 = jnp.concatenate([query, qc[:, :, None]], axis=-1)
    r4 = jnp.concatenate([jnp.transpose(ref, (0, 2, 1)), rc[:, None, :]], axis=1)
    return pl.pallas_call(
        _knn_body(k, masked),
        grid=(b, nq // qb),
        in_specs=[
            pl.BlockSpec((1, qb, 4), lambda i, j: (i, j, 0)),
            pl.BlockSpec((1, 4, nr), lambda i, j: (i, 0, 0)),
        ],
        out_specs=pl.BlockSpec((1, qb, k), lambda i, j: (i, j, 0)),
        out_shape=jax.ShapeDtypeStruct((b, nq, k), jnp.int32),
    )(q4, r4)


def _sc_gather(table, gidx):
    r, d = table.shape
    m = gidx.shape[0]
    m_per_w = m // NW
    ch = min(128, m_per_w)
    n_chunks = m_per_w // ch
    mesh = plsc.VectorSubcoreMesh(core_axis_name="c", subcore_axis_name="s")

    @functools.partial(
        pl.kernel,
        mesh=mesh,
        out_type=jax.ShapeDtypeStruct((m, d), jnp.float32),
        scratch_types=[
            pltpu.VMEM((ch,), jnp.int32),
            pltpu.VMEM((ch, d), jnp.float32),
            pltpu.SemaphoreType.DMA,
        ],
    )
    def k(table_hbm, idx_hbm, out_hbm, idx_v, rows_v, sem):
        wid = lax.axis_index("s") * 2 + lax.axis_index("c")
        base = wid * m_per_w

        def chunk(i, carry):
            off = base + i * ch
            pltpu.sync_copy(idx_hbm.at[pl.ds(off, ch)], idx_v)
            pltpu.async_copy(table_hbm.at[idx_v], rows_v, sem).wait()
            pltpu.sync_copy(rows_v, out_hbm.at[pl.ds(off, ch)])
            return carry

        lax.fori_loop(0, n_chunks, chunk, 0)

    return k(table, gidx)


def _stage_a_body(xc_ref, ft_ref, w_ref, b_ref, out_ref):
    ft = ft_ref[0]
    nb = ft.shape[1]
    feats = jax.nn.relu(
        lax.dot_general(ft, w_ref[...], (((0,), (0,)), ((), ())),
                        preferred_element_type=jnp.float32)
        + b_ref[...][None, :]
    )
    pad = jnp.zeros((nb, 128 - 16 - C1), jnp.float32)
    out_ref[0] = jnp.concatenate([xc_ref[0], feats, pad], axis=1)


def _stage_a(xyzc, features, W1dT, b1d):
    nb = 2048
    return pl.pallas_call(
        _stage_a_body,
        grid=(B, N // nb),
        in_specs=[
            pl.BlockSpec((1, nb, 16), lambda i, j: (i, j, 0)),
            pl.BlockSpec((1, C_IN, nb), lambda i, j: (i, 0, j)),
            pl.BlockSpec((C_IN, C1), lambda i, j: (0, 0)),
            pl.BlockSpec((C1,), lambda i, j: (0,)),
        ],
        out_specs=pl.BlockSpec((1, nb, 128), lambda i, j: (i, j, 0)),
        out_shape=jax.ShapeDtypeStruct((B, N, 128), jnp.float32),
    )(xyzc, features, W1dT, b1d)


def _group_stage_body(qb, k, cg, dd, cv):
    def body(g_ref, q_ref, w_ref, b_ref, t_ref, c_ref):
        g = g_ref[0, 0]
        q = q_ref[0, 0]
        qe = jnp.broadcast_to(q[:, None, :], (qb, k, 128)).reshape(qb * k, 128)
        diff = g - qe
        act = jax.nn.relu(
            jnp.dot(diff, w_ref[...], preferred_element_type=jnp.float32)
            + b_ref[...][None, :]
        )
        w = act[:, :cg]
        val = jnp.concatenate([act[:, cg:cg + dd], g[:, 16:16 + cv]], axis=1)
        w3 = w.reshape(qb, k, cg)
        val3 = val.reshape(qb, k, cg)
        mx = jnp.max(w3, axis=1)
        e = jnp.exp(w3 - mx[:, None, :])
        s = jnp.sum(e, axis=1)
        gfea = jnp.sum(val3 * (e / s[:, None, :]), axis=1)
        pad = jnp.zeros((qb, 13), jnp.float32)
        pad2 = jnp.zeros((qb, 128 - 16 - cg), jnp.float32)
        t_ref[0, 0] = jnp.concatenate([q[:, 0:3], pad, gfea, pad2], axis=1)
        c_ref[0, 0] = mx

    return body


def _group_stage(grows, qrows, wcat, bcat, k, cg, dd, cv, qb=128):
    nq = NPOINT
    nblk = nq // qb
    g4 = grows.reshape(B, nblk, qb * k, 128)
    q4 = qrows.reshape(B, nblk, qb, 128)
    tab, cfea = pl.pallas_call(
        _group_stage_body(qb, k, cg, dd, cv),
        grid=(B, nblk),
        in_specs=[
            pl.BlockSpec((1, 1, qb * k, 128), lambda i, j: (i, j, 0, 0)),
            pl.BlockSpec((1, 1, qb, 128), lambda i, j: (i, j, 0, 0)),
            pl.BlockSpec((128, cg + dd), lambda i, j: (0, 0)),
            pl.BlockSpec((cg + dd,), lambda i, j: (0,)),
        ],
        out_specs=[
            pl.BlockSpec((1, 1, qb, 128), lambda i, j: (i, j, 0, 0)),
            pl.BlockSpec((1, 1, qb, cg), lambda i, j: (i, j, 0, 0)),
        ],
        out_shape=[
            jax.ShapeDtypeStruct((B, nblk, qb, 128), jnp.float32),
            jax.ShapeDtypeStruct((B, nblk, qb, cg), jnp.float32),
        ],
    )(g4, q4, wcat, bcat)
    return tab.reshape(B, nq, 128), cfea.reshape(B, nq, cg)


def _final_stage_body(qb, k):
    def body(g_ref, q_ref, wsp_ref, bsp_ref, c2_ref, t2_ref, c1_ref, q1_ref,
             wf_ref, bn_ref, out_ref):
        g = g_ref[0, 0]
        q = q_ref[0, 0]
        qe = jnp.broadcast_to(q[:, None, :], (qb, k, 128)).reshape(qb * k, 128)
        diff = g - qe
        act = jax.nn.relu(
            jnp.dot(diff, wsp_ref[...], preferred_element_type=jnp.float32)
            + bsp_ref[...][None, :]
        )
        local = jnp.max(act.reshape(qb, k, CP), axis=1)
        c2 = c2_ref[0, 0]
        gf2 = q[:, 16:112]
        gf1 = t2_ref[0, 0][:, 16:96]
        c1 = c1_ref[0, 0]
        ctr = q1_ref[0, 0][:, 16:80]
        fea = jnp.concatenate([c2, local, gf2, gf1, c1, ctr], axis=1)
        out_ref[0, 0] = jax.nn.relu(
            jnp.dot(fea, wf_ref[...], preferred_element_type=jnp.float32)
            + bn_ref[...][None, :]
        )

    return body


def _final_stage(grows4, table4, Wsp3pT, bsp3, c_fea2, table2, c_fea1, qrows1,
                 Wfin, bnew, qb=128):
    nblk = NPOINT // qb
    g4 = grows4.reshape(B, nblk, qb * K4, 128)
    return pl.pallas_call(
        _final_stage_body(qb, K4),
        grid=(B, nblk),
        in_specs=[
            pl.BlockSpec((1, 1, qb * K4, 128), lambda i, j: (i, j, 0, 0)),
            pl.BlockSpec((1, 1, qb, 128), lambda i, j: (i, j, 0, 0)),
            pl.BlockSpec((128, CP), lambda i, j: (0, 0)),
            pl.BlockSpec((CP,), lambda i, j: (0,)),
            pl.BlockSpec((1, 1, qb, 96), lambda i, j: (i, j, 0, 0)),
            pl.BlockSpec((1, 1, qb, 128), lambda i, j: (i, j, 0, 0)),
            pl.BlockSpec((1, 1, qb, 80), lambda i, j: (i, j, 0, 0)),
            pl.BlockSpec((1, 1, qb, 128), lambda i, j: (i, j, 0, 0)),
            pl.BlockSpec((480, C_OUT), lambda i, j: (0, 0)),
            pl.BlockSpec((C_OUT,), lambda i, j: (0,)),
        ],
        out_specs=pl.BlockSpec((1, 1, qb, C_OUT), lambda i, j: (i, j, 0, 0)),
        out_shape=jax.ShapeDtypeStruct((B, nblk, qb, C_OUT), jnp.float32),
    )(
        g4,
        table4.reshape(B, nblk, qb, 128),
        Wsp3pT, bsp3,
        c_fea2.reshape(B, nblk, qb, 96),
        table2.reshape(B, nblk, qb, 128),
        c_fea1.reshape(B, nblk, qb, 80),
        qrows1.reshape(B, nblk, qb, 128),
        Wfin, bnew,
    ).reshape(B, NPOINT, C_OUT)


def _pad_w(w, cols_xyz, cols_fea, width):
    o = w.shape[0]
    out = jnp.zeros((o, width), jnp.float32)
    out = out.at[:, 0:3].set(w[:, 0:3])
    out = out.at[:, 16:16 + cols_fea].set(w[:, 3:3 + cols_fea])
    return out


def kernel(xyz, features, comp, W1d, b1d, Wdx1, bdx1, Ww1, bw1, Wdx2, bdx2,
           Ww2, bw2, Ww3, bw3, Wsp3, bsp3, Wnew, bnew):
    xyz_sg = lax.stop_gradient(xyz)

    xyzc = jnp.concatenate(
        [xyz, comp.astype(jnp.float32)[:, :, None],
         jnp.zeros((B, N, 12), jnp.float32)], axis=-1)

    WcatB = jnp.concatenate(
        [_pad_w(Ww1, 3, C1, 128), _pad_w(Wdx1, 3, 0, 128)], axis=0).T
    bcatB = jnp.concatenate([bw1, bdx1])
    WcatC = jnp.concatenate(
        [_pad_w(Ww2, 3, CG1, 128), _pad_w(Wdx2, 3, 0, 128)], axis=0).T
    bcatC = jnp.concatenate([bw2, bdx2])
    Wsp3pT = _pad_w(Wsp3, 3, CW2, 128).T
    WnewT = Wnew.T
    Wfin = jnp.concatenate([
        WnewT[0:96] + WnewT[256:352],
        WnewT[96:160],
        WnewT[160:256],
        WnewT[352:432],
        WnewT[432:512],
        WnewT[512:576],
    ], axis=0)

    table1 = _stage_a(xyzc, features, W1d.T, b1d)
    table1f = table1.reshape(B * N, 128)

    cidx = _fps(xyz_sg, NPOINT)
    qrows1 = _sc_gather(table1f, cidx.reshape(-1)).reshape(B, NPOINT, 128)
    new_xyz = qrows1[:, :, 0:3]
    new_comp = qrows1[:, :, 3].astype(jnp.int32)

    idx1 = _knn(new_xyz, xyz_sg, K1)
    idx2 = _knn(new_xyz, new_xyz, K2, new_comp, new_comp)

    rows1 = _sc_gather(table1f, idx1.reshape(-1))
    table2, c_fea1 = _group_stage(
        rows1, qrows1, WcatB, bcatB, K1, CG1, D1, C1)

    gidx2 = idx2.reshape(-1)
    rows2 = _sc_gather(table2.reshape(B * NPOINT, 128), gidx2)
    table4, c_fea2 = _group_stage(
        rows2, table2, WcatC, bcatC, K2, CW2, D2, CG1)

    rows4 = _sc_gather(table4.reshape(B * NPOINT, 128), gidx2)
    out = _final_stage(rows4, table4, Wsp3pT, bsp3, c_fea2, table2, c_fea1,
                       qrows1, Wfin, bnew)

    new_features = jnp.transpose(out, (0, 2, 1))
    return new_xyz, new_features, new_comp


# speedup vs baseline: 1.0166x; 1.0166x over previous
import functools

import jax
import jax.numpy as jnp
from jax import lax
from jax.experimental import pallas as pl
from jax.experimental.pallas import tpu as pltpu
from jax.experimental.pallas import tpu_sc as plsc

B, N, NPOINT, SP_NUM = 2, 8192, 1024, 64
K1, K2, K3, K4 = 32, 16, 32, 16
C_IN, C1, D1, D2, CW2, CP, C_OUT = 32, 64, 16, 16, 96, 64, 128
CG1 = D1 + C1

NW = 32


def _fps_kernel_body(npoint, rows, cols):
    irows = max(npoint // 128, 1)

    def body(xyz_ref, out_ref):
        v = xyz_ref[...]
        flat = (
            lax.broadcasted_iota(jnp.int32, (rows, cols), 0) * cols
            + lax.broadcasted_iota(jnp.int32, (rows, cols), 1)
        )[None]
        pos = (
            lax.broadcasted_iota(jnp.int32, (irows, 128), 0) * 128
            + lax.broadcasted_iota(jnp.int32, (irows, 128), 1)
        )[None]

        def step(i, state):
            idxs, dists, far = state
            idxs = jnp.where(pos == i, far, idxs)
            sel = flat == far
            fxyz = jnp.sum(
                jnp.where(sel[:, None], v, 0.0), axis=(2, 3), keepdims=True
            )
            d = (
                (v[:, 0] - fxyz[:, 0]) ** 2
                + (v[:, 1] - fxyz[:, 1]) ** 2
                + (v[:, 2] - fxyz[:, 2]) ** 2
            )
            dists = jnp.minimum(dists, d)
            far2 = jnp.argmax(
                dists.reshape(B, rows * cols), axis=1
            ).astype(jnp.int32)[:, None, None]
            return (idxs, dists, far2)

        idxs0 = jnp.zeros((B, irows, 128), jnp.int32)
        d0 = jnp.full((B, rows, cols), 1e10, jnp.float32)
        far0 = jnp.zeros((B, 1, 1), jnp.int32)
        idxs, _, _ = lax.fori_loop(0, npoint, step, (idxs0, d0, far0))
        boff = lax.broadcasted_iota(jnp.int32, (B, irows, 128), 0) * (rows * cols)
        out_ref[...] = idxs + boff

    return body, irows


def _fps(xyz, npoint):
    n = xyz.shape[1]
    cols = 1024 if n >= 8192 else 128
    rows = n // cols
    xyz_t = jnp.transpose(xyz, (0, 2, 1)).reshape(B, 3, rows, cols)
    body, irows = _fps_kernel_body(npoint, rows, cols)
    out = pl.pallas_call(
        body,
        grid=(1,),
        in_specs=[pl.BlockSpec((B, 3, rows, cols), lambda i: (0, 0, 0, 0))],
        out_specs=pl.BlockSpec((B, irows, 128), lambda i: (0, 0, 0)),
        out_shape=jax.ShapeDtypeStruct((B, irows, 128), jnp.int32),
    )(xyz_t)
    return out.reshape(B, irows * 128)[:, :npoint]


def _knn_body(k, masked):
    def body(q_ref, r_ref, out_ref):
        q = q_ref[0]
        qb = q.shape[0]
        nr = r_ref.shape[2]
        rx = r_ref[0, 0:1, :]
        ry = r_ref[0, 1:2, :]
        rz = r_ref[0, 2:3, :]
        d = (q[:, 0:1] - rx) ** 2 + (q[:, 1:2] - ry) ** 2 + (q[:, 2:3] - rz) ** 2
        if masked:
            rc = r_ref[0, 3:4, :]
            d = d + 1e9 * (q[:, 3:4] != rc).astype(jnp.float32)
        col = lax.broadcasted_iota(jnp.int32, (qb, nr), 1)
        kcol = lax.broadcasted_iota(jnp.int32, (qb, k), 1)

        def ext(j, state):
            dd, idxs, prev = state
            dd = jnp.where(col == prev, jnp.inf, dd)
            idx = jnp.argmin(dd, axis=1).astype(jnp.int32)[:, None]
            idxs = jnp.where(kcol == j, idx, idxs)
            return dd, idxs, idx

        _, idxs, _ = lax.fori_loop(
            0, k, ext,
            (d, jnp.zeros((qb, k), jnp.int32), jnp.full((qb, 1), -1, jnp.int32)))
        out_ref[0] = idxs + pl.program_id(0) * nr

    return body


def _knn(query, ref, k, qcomp=None, rcomp=None, qb=64):
    b, nq, _ = query.shape
    nr = ref.shape[1]
    masked = qcomp is not None
    qc = qcomp.astype(jnp.float32) if masked else jnp.zeros((b, nq), jnp.float32)
    rc = rcomp.astype(jnp.float32) if masked else jnp.zeros((b, nr), jnp.float32)
    q4 = jnp.concatenate([query, qc[:, :, None]], axis=-1)
    r4 = jnp.concatenate([jnp.transpose(ref, (0, 2, 1)), rc[:, None, :]], axis=1)
    return pl.pallas_call(
        _knn_body(k, masked),
        grid=(b, nq // qb),
        in_specs=[
            pl.BlockSpec((1, qb, 4), lambda i, j: (i, j, 0)),
            pl.BlockSpec((1, 4, nr), lambda i, j: (i, 0, 0)),
        ],
        out_specs=pl.BlockSpec((1, qb, k), lambda i, j: (i, j, 0)),
        out_shape=jax.ShapeDtypeStruct((b, nq, k), jnp.int32),
    )(q4, r4)


def _sc_gather(table, gidx):
    r, d = table.shape
    m = gidx.shape[0]
    m_per_w = m // NW
    ch = min(128, m_per_w)
    n_chunks = m_per_w // ch
    mesh = plsc.VectorSubcoreMesh(core_axis_name="c", subcore_axis_name="s")

    @functools.partial(
        pl.kernel,
        mesh=mesh,
        out_type=jax.ShapeDtypeStruct((m, d), jnp.float32),
        scratch_types=[
            pltpu.VMEM((ch,), jnp.int32),
            pltpu.VMEM((ch, d), jnp.float32),
            pltpu.SemaphoreType.DMA,
        ],
    )
    def k(table_hbm, idx_hbm, out_hbm, idx_v, rows_v, sem):
        wid = lax.axis_index("s") * 2 + lax.axis_index("c")
        base = wid * m_per_w

        def chunk(i, carry):
            off = base + i * ch
            pltpu.sync_copy(idx_hbm.at[pl.ds(off, ch)], idx_v)
            pltpu.async_copy(table_hbm.at[idx_v], rows_v, sem).wait()
            pltpu.sync_copy(rows_v, out_hbm.at[pl.ds(off, ch)])
            return carry

        lax.fori_loop(0, n_chunks, chunk, 0)

    return k(table, gidx)


def _stage_a_body(xc_ref, ft_ref, w_ref, b_ref, out_ref):
    ft = ft_ref[0]
    nb = ft.shape[1]
    feats = jax.nn.relu(
        lax.dot_general(ft, w_ref[...], (((0,), (0,)), ((), ())),
                        preferred_element_type=jnp.float32)
        + b_ref[...][None, :]
    )
    pad = jnp.zeros((nb, 128 - 16 - C1), jnp.float32)
    out_ref[0] = jnp.concatenate([xc_ref[0], feats, pad], axis=1)


def _stage_a(xyzc, features, W1dT, b1d):
    nb = 2048
    return pl.pallas_call(
        _stage_a_body,
        grid=(B, N // nb),
        in_specs=[
            pl.BlockSpec((1, nb, 16), lambda i, j: (i, j, 0)),
            pl.BlockSpec((1, C_IN, nb), lambda i, j: (i, 0, j)),
            pl.BlockSpec((C_IN, C1), lambda i, j: (0, 0)),
            pl.BlockSpec((C1,), lambda i, j: (0,)),
        ],
        out_specs=pl.BlockSpec((1, nb, 128), lambda i, j: (i, j, 0)),
        out_shape=jax.ShapeDtypeStruct((B, N, 128), jnp.float32),
    )(xyzc, features, W1dT, b1d)


def _group_stage_body(qb, k, cg, dd, cv):
    def body(g_ref, q_ref, w_ref, b_ref, t_ref, c_ref):
        g = g_ref[0, 0]
        q = q_ref[0, 0]
        qe = jnp.broadcast_to(q[:, None, :], (qb, k, 128)).reshape(qb * k, 128)
        diff = g - qe
        act = jax.nn.relu(
            jnp.dot(diff, w_ref[...], preferred_element_type=jnp.float32)
            + b_ref[...][None, :]
        )
        w = act[:, :cg]
        val = jnp.concatenate([act[:, cg:cg + dd], g[:, 16:16 + cv]], axis=1)
        w3 = w.reshape(qb, k, cg)
        val3 = val.reshape(qb, k, cg)
        mx = jnp.max(w3, axis=1)
        e = jnp.exp(w3 - mx[:, None, :])
        s = jnp.sum(e, axis=1)
        gfea = jnp.sum(val3 * (e / s[:, None, :]), axis=1)
        pad = jnp.zeros((qb, 13), jnp.float32)
        pad2 = jnp.zeros((qb, 128 - 16 - cg), jnp.float32)
        t_ref[0, 0] = jnp.concatenate([q[:, 0:3], pad, gfea, pad2], axis=1)
        c_ref[0, 0] = mx

    return body


def _group_stage(grows, qrows, wcat, bcat, k, cg, dd, cv, qb=128):
    nq = NPOINT
    nblk = nq // qb
    g4 = grows.reshape(B, nblk, qb * k, 128)
    q4 = qrows.reshape(B, nblk, qb, 128)
    tab, cfea = pl.pallas_call(
        _group_stage_body(qb, k, cg, dd, cv),
        grid=(B, nblk),
        in_specs=[
            pl.BlockSpec((1, 1, qb * k, 128), lambda i, j: (i, j, 0, 0)),
            pl.BlockSpec((1, 1, qb, 128), lambda i, j: (i, j, 0, 0)),
            pl.BlockSpec((128, cg + dd), lambda i, j: (0, 0)),
            pl.BlockSpec((cg + dd,), lambda i, j: (0,)),
        ],
        out_specs=[
            pl.BlockSpec((1, 1, qb, 128), lambda i, j: (i, j, 0, 0)),
            pl.BlockSpec((1, 1, qb, cg), lambda i, j: (i, j, 0, 0)),
        ],
        out_shape=[
            jax.ShapeDtypeStruct((B, nblk, qb, 128), jnp.float32),
            jax.ShapeDtypeStruct((B, nblk, qb, cg), jnp.float32),
        ],
    )(g4, q4, wcat, bcat)
    return tab.reshape(B, nq, 128), cfea.reshape(B, nq, cg)


def _final_stage_body(qb, k):
    def body(g_ref, q_ref, wsp_ref, bsp_ref, c2_ref, t2_ref, c1_ref, q1_ref,
             wf_ref, bn_ref, out_ref):
        g = g_ref[0, 0]
        q = q_ref[0, 0]
        qe = jnp.broadcast_to(q[:, None, :], (qb, k, 128)).reshape(qb * k, 128)
        diff = g - qe
        act = jax.nn.relu(
            jnp.dot(diff, wsp_ref[...], preferred_element_type=jnp.float32)
            + bsp_ref[...][None, :]
        )
        local = jnp.max(act.reshape(qb, k, CP), axis=1)
        c2 = c2_ref[0, 0]
        gf2 = q[:, 16:112]
        gf1 = t2_ref[0, 0][:, 16:96]
        c1 = c1_ref[0, 0]
        ctr = q1_ref[0, 0][:, 16:80]
        fea = jnp.concatenate([c2, local, gf2, gf1, c1, ctr], axis=1)
        res = jax.nn.relu(
            jnp.dot(fea, wf_ref[...], preferred_element_type=jnp.float32)
            + bn_ref[...][None, :]
        )
        out_ref[0] = res.T

    return body


def _final_stage(grows4, table4, Wsp3pT, bsp3, c_fea2, table2, c_fea1, qrows1,
                 Wfin, bnew, qb=128):
    nblk = NPOINT // qb
    g4 = grows4.reshape(B, nblk, qb * K4, 128)
    return pl.pallas_call(
        _final_stage_body(qb, K4),
        grid=(B, nblk),
        in_specs=[
            pl.BlockSpec((1, 1, qb * K4, 128), lambda i, j: (i, j, 0, 0)),
            pl.BlockSpec((1, 1, qb, 128), lambda i, j: (i, j, 0, 0)),
            pl.BlockSpec((128, CP), lambda i, j: (0, 0)),
            pl.BlockSpec((CP,), lambda i, j: (0,)),
            pl.BlockSpec((1, 1, qb, 96), lambda i, j: (i, j, 0, 0)),
            pl.BlockSpec((1, 1, qb, 128), lambda i, j: (i, j, 0, 0)),
            pl.BlockSpec((1, 1, qb, 80), lambda i, j: (i, j, 0, 0)),
            pl.BlockSpec((1, 1, qb, 128), lambda i, j: (i, j, 0, 0)),
            pl.BlockSpec((480, C_OUT), lambda i, j: (0, 0)),
            pl.BlockSpec((C_OUT,), lambda i, j: (0,)),
        ],
        out_specs=pl.BlockSpec((1, C_OUT, qb), lambda i, j: (i, 0, j)),
        out_shape=jax.ShapeDtypeStruct((B, C_OUT, NPOINT), jnp.float32),
    )(
        g4,
        table4.reshape(B, nblk, qb, 128),
        Wsp3pT, bsp3,
        c_fea2.reshape(B, nblk, qb, 96),
        table2.reshape(B, nblk, qb, 128),
        c_fea1.reshape(B, nblk, qb, 80),
        qrows1.reshape(B, nblk, qb, 128),
        Wfin, bnew,
    )


def _pad_w(w, cols_xyz, cols_fea, width):
    o = w.shape[0]
    out = jnp.zeros((o, width), jnp.float32)
    out = out.at[:, 0:3].set(w[:, 0:3])
    out = out.at[:, 16:16 + cols_fea].set(w[:, 3:3 + cols_fea])
    return out


def kernel(xyz, features, comp, W1d, b1d, Wdx1, bdx1, Ww1, bw1, Wdx2, bdx2,
           Ww2, bw2, Ww3, bw3, Wsp3, bsp3, Wnew, bnew):
    xyz_sg = lax.stop_gradient(xyz)

    xyzc = jnp.concatenate(
        [xyz, comp.astype(jnp.float32)[:, :, None],
         jnp.zeros((B, N, 12), jnp.float32)], axis=-1)

    WcatB = jnp.concatenate(
        [_pad_w(Ww1, 3, C1, 128), _pad_w(Wdx1, 3, 0, 128)], axis=0).T
    bcatB = jnp.concatenate([bw1, bdx1])
    WcatC = jnp.concatenate(
        [_pad_w(Ww2, 3, CG1, 128), _pad_w(Wdx2, 3, 0, 128)], axis=0).T
    bcatC = jnp.concatenate([bw2, bdx2])
    Wsp3pT = _pad_w(Wsp3, 3, CW2, 128).T
    WnewT = Wnew.T
    Wfin = jnp.concatenate([
        WnewT[0:96] + WnewT[256:352],
        WnewT[96:160],
        WnewT[160:256],
        WnewT[352:432],
        WnewT[432:512],
        WnewT[512:576],
    ], axis=0)

    table1 = _stage_a(xyzc, features, W1d.T, b1d)
    table1f = table1.reshape(B * N, 128)

    cidx = _fps(xyz_sg, NPOINT)
    qrows1 = _sc_gather(table1f, cidx.reshape(-1)).reshape(B, NPOINT, 128)
    new_xyz = qrows1[:, :, 0:3]
    new_comp = qrows1[:, :, 3].astype(jnp.int32)

    idx1 = _knn(new_xyz, xyz_sg, K1)
    idx2 = _knn(new_xyz, new_xyz, K2, new_comp, new_comp)

    rows1 = _sc_gather(table1f, idx1.reshape(-1))
    table2, c_fea1 = _group_stage(
        rows1, qrows1, WcatB, bcatB, K1, CG1, D1, C1)

    gidx2 = idx2.reshape(-1)
    rows2 = _sc_gather(table2.reshape(B * NPOINT, 128), gidx2)
    table4, c_fea2 = _group_stage(
        rows2, table2, WcatC, bcatC, K2, CW2, D2, CG1)

    rows4 = _sc_gather(table4.reshape(B * NPOINT, 128), gidx2)
    new_features = _final_stage(rows4, table4, Wsp3pT, bsp3, c_fea2, table2,
                                c_fea1, qrows1, Wfin, bnew)
    return new_xyz, new_features, new_comp
